# baseline (speedup 1.0000x reference)
import jax
import jax.numpy as jnp
from jax.experimental import pallas as pl
from jax.experimental.pallas import tpu as pltpu

S, D = 2048, 2048
H, KV, HD = 16, 8, 128
G = H // KV
BLK = 64
NB = S // BLK
BUDGET = 16
SWIN = 4
SCALE = HD ** -0.5
EPS = 1e-6
NEG = -1e30

TQ = 512
QT = S // TQ
GB = TQ // BLK

CH = 512

PREC = jax.lax.Precision.DEFAULT
HI = jax.lax.Precision.HIGHEST


def _iota(shape, dim):
    return jax.lax.broadcasted_iota(jnp.int32, shape, dim)


def _dot32(a, b):
    return jnp.dot(a, b, precision=PREC, preferred_element_type=jnp.float32)


def _split_dot_xi(x, ind_bf):
    xh = x.astype(jnp.bfloat16)
    xl = (x - xh.astype(jnp.float32)).astype(jnp.bfloat16)
    return (jnp.dot(xh, ind_bf, preferred_element_type=jnp.float32)
            + jnp.dot(xl, ind_bf, preferred_element_type=jnp.float32))


def _split_dot_ix(ind_bf, x):
    xh = x.astype(jnp.bfloat16)
    xl = (x - xh.astype(jnp.float32)).astype(jnp.bfloat16)
    return (jnp.dot(ind_bf, xh, preferred_element_type=jnp.float32)
            + jnp.dot(ind_bf, xl, preferred_element_type=jnp.float32))


def _pnr_kernel(x_ref, w_ref, cs_ref, sn_ref, nw_ref, o_ref):
    y = jnp.dot(x_ref[...], w_ref[...], precision=PREC,
                preferred_element_type=jnp.float32)
    cs = cs_ref[...]
    sn = sn_ref[...]
    nw = nw_ref[...]
    cols = []
    for h in range(y.shape[1] // HD):
        yh = y[:, h * HD:(h + 1) * HD]
        var = jnp.mean(yh * yh, axis=-1, keepdims=True)
        n = (yh * jax.lax.rsqrt(var + EPS)) * nw
        rot = jnp.concatenate([-n[:, HD // 2:], n[:, :HD // 2]], axis=-1)
        cols.append(n * cs + rot * sn)
    o_ref[...] = jnp.concatenate(cols, axis=-1)


def _mm_kernel(x_ref, w_ref, o_ref):
    o_ref[...] = jnp.dot(x_ref[...], w_ref[...], precision=PREC,
                         preferred_element_type=jnp.float32)


def _seer_kernel(q_ref, k_ref, v_ref, o_ref, e0_scr, e1_scr):
    qt = pl.program_id(1)
    diag = _iota((TQ, CH), 0) >= _iota((TQ, CH), 1)
    rq_t = (_iota((GB, TQ), 1) // BLK
            == _iota((GB, TQ), 0)).astype(jnp.bfloat16)
    rq = (_iota((TQ, GB), 0) // BLK == _iota((TQ, GB), 1)).astype(jnp.float32)
    q0 = q_ref[:, :HD]
    q1 = q_ref[:, HD:]

    def _chunk(c, carry, masked):
        a0, a1 = carry
        kc = k_ref[pl.ds(c * CH, CH), :]
        rk_c = (((c * CH + _iota((CH, NB), 0)) // BLK)
                == _iota((CH, NB), 1)).astype(jnp.bfloat16)
        s0 = jax.lax.dot_general(
            q0, kc, (((1,), (1,)), ((), ())), precision=PREC,
            preferred_element_type=jnp.float32) * SCALE
        s1 = jax.lax.dot_general(
            q1, kc, (((1,), (1,)), ((), ())), precision=PREC,
            preferred_element_type=jnp.float32) * SCALE
        if masked:
            s0 = jnp.where(diag, s0, NEG)
            s1 = jnp.where(diag, s1, NEG)
        e0 = jnp.exp(s0)
        e1 = jnp.exp(s1)
        e0_scr[c] = e0
        e1_scr[c] = e1
        a0 = a0 + _split_dot_xi(e0, rk_c)
        a1 = a1 + _split_dot_xi(e1, rk_c)
        return a0, a1

    z_a = jnp.zeros((TQ, NB), jnp.float32)
    carry = jax.lax.fori_loop(0, qt, lambda c, cy: _chunk(c, cy, False),
                              (z_a, z_a))
    a0, a1 = _chunk(qt, carry, True)
    l0 = jnp.sum(a0, axis=-1, keepdims=True)
    l1 = jnp.sum(a1, axis=-1, keepdims=True)
    p0 = _split_dot_ix(rq_t, a0 / l0)
    p1 = _split_dot_ix(rq_t, a1 / l1)
    pooled = jnp.maximum(p0, p1)
    qb = qt * GB + _iota((GB, NB), 0)
    kb = _iota((GB, NB), 1)
    pc = jnp.where(kb <= qb, pooled, -1.0)
    cnt = jnp.zeros((GB, NB), jnp.float32)
    one = jnp.ones((GB, NB), jnp.float32)
    zero = jnp.zeros((GB, NB), jnp.float32)
    for shift in range(1, NB):
        r = jnp.roll(pc, shift, axis=1)
        cnt += jnp.where(r > pc, one, zero)
        cnt += jnp.where((r == pc) & (kb >= shift), one, zero)
    sel = cnt < BUDGET
    window = ((qb - kb) < SWIN) & (kb <= qb)
    allowed = sel | window | (kb == 0)
    b01 = jnp.where(allowed, 1.0, 0.0)
    b01e = _dot32(rq, b01)
    l20 = jnp.sum(a0 * b01e, axis=-1, keepdims=True)
    l21 = jnp.sum(a1 * b01e, axis=-1, keepdims=True)

    def body2(c, carry):
        o0, o1 = carry
        vc = v_ref[pl.ds(c * CH, CH), :]
        rkt_c = (_iota((NB, CH), 0)
                 == (c * CH + _iota((NB, CH), 1)) // BLK).astype(jnp.float32)
        bm = _dot32(rq, _dot32(b01, rkt_c))
        o0 = o0 + _dot32(e0_scr[c] * bm, vc)
        o1 = o1 + _dot32(e1_scr[c] * bm, vc)
        return o0, o1

    z_o = jnp.zeros((TQ, HD), jnp.float32)
    o0, o1 = jax.lax.fori_loop(0, qt + 1, body2, (z_o, z_o))
    o_ref[...] = jnp.concatenate([o0 / l20, o1 / l21], axis=-1)


def _pnr_call(x, w, cs, sn, nw, n_ct):
    st = S // TQ
    return pl.pallas_call(
        _pnr_kernel,
        grid=(n_ct, st),
        in_specs=[
            pl.BlockSpec((TQ, D), lambda ct, s: (s, 0)),
            pl.BlockSpec((D, 1024), lambda ct, s: (0, ct)),
            pl.BlockSpec((TQ, HD), lambda ct, s: (s, 0)),
            pl.BlockSpec((TQ, HD), lambda ct, s: (s, 0)),
            pl.BlockSpec((1, HD), lambda ct, s: (0, 0)),
        ],
        out_specs=pl.BlockSpec((TQ, 1024), lambda ct, s: (s, ct)),
        out_shape=jax.ShapeDtypeStruct((S, n_ct * 1024), jnp.float32),
    )(x, w, cs, sn, nw)


def _mm_call(x, w, bm, bn):
    gm, gn = x.shape[0] // bm, w.shape[1] // bn
    return pl.pallas_call(
        _mm_kernel,
        grid=(gn, gm),
        in_specs=[
            pl.BlockSpec((bm, x.shape[1]), lambda n, m: (m, 0)),
            pl.BlockSpec((w.shape[0], bn), lambda n, m: (0, n)),
        ],
        out_specs=pl.BlockSpec((bm, bn), lambda n, m: (m, n)),
        out_shape=jax.ShapeDtypeStruct((x.shape[0], w.shape[1]), jnp.float32),
    )(x, w)


def _seer_call(q, k, v):
    return pl.pallas_call(
        _seer_kernel,
        grid=(KV, QT),
        in_specs=[
            pl.BlockSpec((TQ, G * HD), lambda j, qt: (qt, j)),
            pl.BlockSpec((S, HD), lambda j, qt: (0, j)),
            pl.BlockSpec((S, HD), lambda j, qt: (0, j)),
        ],
        out_specs=pl.BlockSpec((TQ, G * HD), lambda j, qt: (qt, j)),
        out_shape=jax.ShapeDtypeStruct((S, H * HD), jnp.float32),
        scratch_shapes=[
            pltpu.VMEM((QT, TQ, CH), jnp.float32),
            pltpu.VMEM((QT, TQ, CH), jnp.float32),
        ],
    )(q, k, v)


def kernel(hidden_states, cos, sin, Wq, Wk, Wv, Wo, q_norm_w, k_norm_w):
    x = hidden_states[0]
    cs = cos[0]
    sn = sin[0]
    qn = q_norm_w.reshape(1, HD)
    kn = k_norm_w.reshape(1, HD)
    q = _pnr_call(x, Wq, cs, sn, qn, 2)
    k = _pnr_call(x, Wk, cs, sn, kn, 1)
    v = _mm_call(x, Wv, 512, 1024)
    att = _seer_call(q, k, v)
    out = _mm_call(att, Wo, 1024, 1024)
    return out[None]


# speedup vs baseline: 1.0899x; 1.0899x over previous
import jax
import jax.numpy as jnp
from jax.experimental import pallas as pl
from jax.experimental.pallas import tpu as pltpu

S, D = 2048, 2048
H, KV, HD = 16, 8, 128
G = H // KV
BLK = 64
NB = S // BLK
BUDGET = 16
SWIN = 4
SCALE = HD ** -0.5
EPS = 1e-6
NEG = -1e30

TQ = 512
QT = S // TQ
GB = TQ // BLK

CH = 512

PREC = jax.lax.Precision.DEFAULT
HI = jax.lax.Precision.HIGHEST


def _iota(shape, dim):
    return jax.lax.broadcasted_iota(jnp.int32, shape, dim)


def _dot32(a, b):
    return jnp.dot(a, b, precision=PREC, preferred_element_type=jnp.float32)


def _split_dot_xi(x, ind_bf):
    xh = x.astype(jnp.bfloat16)
    xl = (x - xh.astype(jnp.float32)).astype(jnp.bfloat16)
    return (jnp.dot(xh, ind_bf, preferred_element_type=jnp.float32)
            + jnp.dot(xl, ind_bf, preferred_element_type=jnp.float32))


def _split_dot_ix(ind_bf, x):
    xh = x.astype(jnp.bfloat16)
    xl = (x - xh.astype(jnp.float32)).astype(jnp.bfloat16)
    return (jnp.dot(ind_bf, xh, preferred_element_type=jnp.float32)
            + jnp.dot(ind_bf, xl, preferred_element_type=jnp.float32))


def _pnr_kernel(x_ref, w_ref, cs_ref, sn_ref, nw_ref, o_ref):
    y = jnp.dot(x_ref[...], w_ref[...], precision=PREC,
                preferred_element_type=jnp.float32)
    cs = cs_ref[...]
    sn = sn_ref[...]
    nw = nw_ref[...]
    cols = []
    for h in range(y.shape[1] // HD):
        yh = y[:, h * HD:(h + 1) * HD]
        var = jnp.mean(yh * yh, axis=-1, keepdims=True)
        n = (yh * jax.lax.rsqrt(var + EPS)) * nw
        rot = jnp.concatenate([-n[:, HD // 2:], n[:, :HD // 2]], axis=-1)
        cols.append(n * cs + rot * sn)
    o_ref[...] = jnp.concatenate(cols, axis=-1)


def _mm_kernel(x_ref, w_ref, o_ref):
    o_ref[...] = jnp.dot(x_ref[...], w_ref[...], precision=PREC,
                         preferred_element_type=jnp.float32)


def _seer_kernel(q_ref, k_ref, v_ref, o_ref, e0_scr, e1_scr, e2_scr, e3_scr):
    escr = (e0_scr, e1_scr, e2_scr, e3_scr)
    qt = pl.program_id(1)
    diag = _iota((TQ, CH), 0) >= _iota((TQ, CH), 1)
    rq_t = (_iota((GB, TQ), 1) // BLK
            == _iota((GB, TQ), 0)).astype(jnp.bfloat16)
    rq = (_iota((TQ, GB), 0) // BLK == _iota((TQ, GB), 1)).astype(jnp.float32)
    qs = [q_ref[:, h * HD:(h + 1) * HD] for h in range(4)]

    def _chunk(c, carry, masked):
        acc = list(carry)
        kc = k_ref[pl.ds(c * CH, CH), :]
        rk_c = (((c * CH + _iota((CH, NB), 0)) // BLK)
                == _iota((CH, NB), 1)).astype(jnp.bfloat16)
        for h in range(4):
            kh = kc[:, (h // 2) * HD:(h // 2 + 1) * HD]
            s = jax.lax.dot_general(
                qs[h], kh, (((1,), (1,)), ((), ())), precision=PREC,
                preferred_element_type=jnp.float32) * SCALE
            if masked:
                s = jnp.where(diag, s, NEG)
            e = jnp.exp(s)
            escr[h][c] = e
            acc[h] = acc[h] + _split_dot_xi(e, rk_c)
        return tuple(acc)

    z_a = jnp.zeros((TQ, NB), jnp.float32)
    carry = jax.lax.fori_loop(0, qt, lambda c, cy: _chunk(c, cy, False),
                              (z_a,) * 4)
    aa = _chunk(qt, carry, True)
    ll = [jnp.sum(a, axis=-1, keepdims=True) for a in aa]
    pp = [_split_dot_ix(rq_t, aa[h] / ll[h]) for h in range(4)]
    pooled = jnp.concatenate([jnp.maximum(pp[0], pp[1]),
                              jnp.maximum(pp[2], pp[3])], axis=0)
    qb = qt * GB + _iota((2 * GB, NB), 0) % GB
    kb = _iota((2 * GB, NB), 1)
    pc = jnp.where(kb <= qb, pooled, -1.0)
    cnt = jnp.zeros((2 * GB, NB), jnp.float32)
    one = jnp.ones((2 * GB, NB), jnp.float32)
    zero = jnp.zeros((2 * GB, NB), jnp.float32)
    for shift in range(1, NB):
        r = jnp.roll(pc, shift, axis=1)
        cnt += jnp.where(r > pc, one, zero)
        cnt += jnp.where((r == pc) & (kb >= shift), one, zero)
    sel = cnt < BUDGET
    window = ((qb - kb) < SWIN) & (kb <= qb)
    allowed = sel | window | (kb == 0)
    b01 = jnp.where(allowed, 1.0, 0.0)
    b01a = b01[:GB]
    b01b = b01[GB:]
    b01ea = _dot32(rq, b01a)
    b01eb = _dot32(rq, b01b)
    l2 = [jnp.sum(aa[0] * b01ea, axis=-1, keepdims=True),
          jnp.sum(aa[1] * b01ea, axis=-1, keepdims=True),
          jnp.sum(aa[2] * b01eb, axis=-1, keepdims=True),
          jnp.sum(aa[3] * b01eb, axis=-1, keepdims=True)]

    def body2(c, carry):
        os_ = list(carry)
        vc = v_ref[pl.ds(c * CH, CH), :]
        rkt_c = (_iota((NB, CH), 0)
                 == (c * CH + _iota((NB, CH), 1)) // BLK).astype(jnp.float32)
        bma = _dot32(rq, _dot32(b01a, rkt_c))
        bmb = _dot32(rq, _dot32(b01b, rkt_c))
        for h in range(4):
            bm = bma if h < 2 else bmb
            vh = vc[:, (h // 2) * HD:(h // 2 + 1) * HD]
            os_[h] = os_[h] + _dot32(escr[h][c] * bm, vh)
        return tuple(os_)

    z_o = jnp.zeros((TQ, HD), jnp.float32)
    oo = jax.lax.fori_loop(0, qt + 1, body2, (z_o,) * 4)
    o_ref[...] = jnp.concatenate([oo[h] / l2[h] for h in range(4)], axis=-1)


def _pnr_call(x, w, cs, sn, nw, n_ct):
    st = S // TQ
    return pl.pallas_call(
        _pnr_kernel,
        grid=(n_ct, st),
        in_specs=[
            pl.BlockSpec((TQ, D), lambda ct, s: (s, 0)),
            pl.BlockSpec((D, 1024), lambda ct, s: (0, ct)),
            pl.BlockSpec((TQ, HD), lambda ct, s: (s, 0)),
            pl.BlockSpec((TQ, HD), lambda ct, s: (s, 0)),
            pl.BlockSpec((1, HD), lambda ct, s: (0, 0)),
        ],
        out_specs=pl.BlockSpec((TQ, 1024), lambda ct, s: (s, ct)),
        out_shape=jax.ShapeDtypeStruct((S, n_ct * 1024), jnp.float32),
    )(x, w, cs, sn, nw)


def _mm_call(x, w, bm, bn):
    gm, gn = x.shape[0] // bm, w.shape[1] // bn
    return pl.pallas_call(
        _mm_kernel,
        grid=(gn, gm),
        in_specs=[
            pl.BlockSpec((bm, x.shape[1]), lambda n, m: (m, 0)),
            pl.BlockSpec((w.shape[0], bn), lambda n, m: (0, n)),
        ],
        out_specs=pl.BlockSpec((bm, bn), lambda n, m: (m, n)),
        out_shape=jax.ShapeDtypeStruct((x.shape[0], w.shape[1]), jnp.float32),
    )(x, w)


def _seer_call(q, k, v):
    return pl.pallas_call(
        _seer_kernel,
        grid=(KV // 2, QT),
        in_specs=[
            pl.BlockSpec((TQ, 4 * HD), lambda j, qt: (qt, j)),
            pl.BlockSpec((S, 2 * HD), lambda j, qt: (0, j)),
            pl.BlockSpec((S, 2 * HD), lambda j, qt: (0, j)),
        ],
        out_specs=pl.BlockSpec((TQ, 4 * HD), lambda j, qt: (qt, j)),
        out_shape=jax.ShapeDtypeStruct((S, H * HD), jnp.float32),
        scratch_shapes=[
            pltpu.VMEM((QT, TQ, CH), jnp.float32),
            pltpu.VMEM((QT, TQ, CH), jnp.float32),
            pltpu.VMEM((QT, TQ, CH), jnp.float32),
            pltpu.VMEM((QT, TQ, CH), jnp.float32),
        ],
    )(q, k, v)


def kernel(hidden_states, cos, sin, Wq, Wk, Wv, Wo, q_norm_w, k_norm_w):
    x = hidden_states[0]
    cs = cos[0]
    sn = sin[0]
    qn = q_norm_w.reshape(1, HD)
    kn = k_norm_w.reshape(1, HD)
    q = _pnr_call(x, Wq, cs, sn, qn, 2)
    k = _pnr_call(x, Wk, cs, sn, kn, 1)
    v = _mm_call(x, Wv, 512, 1024)
    att = _seer_call(q, k, v)
    out = _mm_call(att, Wo, 1024, 1024)
    return out[None]


# speedup vs baseline: 1.2411x; 1.1387x over previous
import jax
import jax.numpy as jnp
from jax.experimental import pallas as pl
from jax.experimental.pallas import tpu as pltpu

S, D = 2048, 2048
H, KV, HD = 16, 8, 128
G = H // KV
BLK = 64
NB = S // BLK
BUDGET = 16
SWIN = 4
SCALE = HD ** -0.5
EPS = 1e-6
NEG = -1e30

TQ = 512
QT = S // TQ
GB = TQ // BLK

CH = 512

PREC = jax.lax.Precision.DEFAULT
HI = jax.lax.Precision.HIGHEST


def _iota(shape, dim):
    return jax.lax.broadcasted_iota(jnp.int32, shape, dim)


def _dot32(a, b):
    return jnp.dot(a, b, precision=PREC, preferred_element_type=jnp.float32)


def _split_dot_xi(x, ind_bf):
    xh = x.astype(jnp.bfloat16)
    xl = (x - xh.astype(jnp.float32)).astype(jnp.bfloat16)
    return (jnp.dot(xh, ind_bf, preferred_element_type=jnp.float32)
            + jnp.dot(xl, ind_bf, preferred_element_type=jnp.float32))


def _split_dot_ix(ind_bf, x):
    xh = x.astype(jnp.bfloat16)
    xl = (x - xh.astype(jnp.float32)).astype(jnp.bfloat16)
    return (jnp.dot(ind_bf, xh, preferred_element_type=jnp.float32)
            + jnp.dot(ind_bf, xl, preferred_element_type=jnp.float32))


def _norm_rope(yh, nw, cs, sn):
    var = jnp.mean(yh * yh, axis=-1, keepdims=True)
    n = (yh * jax.lax.rsqrt(var + EPS)) * nw
    rot = jnp.concatenate([-n[:, HD // 2:], n[:, :HD // 2]], axis=-1)
    return n * cs + rot * sn


def _qkv_kernel(x_ref, wq_ref, wk_ref, wv_ref, cs_ref, sn_ref,
                qn_ref, kn_ref, o_ref):
    x = x_ref[...]
    yq = jnp.dot(x, wq_ref[...], precision=PREC,
                 preferred_element_type=jnp.float32)
    yk = jnp.dot(x, wk_ref[...], precision=PREC,
                 preferred_element_type=jnp.float32)
    yv = jnp.dot(x, wv_ref[...], precision=PREC,
                 preferred_element_type=jnp.float32)
    cs = cs_ref[...]
    sn = sn_ref[...]
    qn = qn_ref[...]
    kn = kn_ref[...]
    cols = []
    for h in range(H):
        cols.append(_norm_rope(yq[:, h * HD:(h + 1) * HD], qn, cs, sn))
    for h in range(KV):
        cols.append(_norm_rope(yk[:, h * HD:(h + 1) * HD], kn, cs, sn))
    cols.append(yv)
    o_ref[...] = jnp.concatenate(cols, axis=-1)


def _mm_kernel(x_ref, w_ref, o_ref):
    o_ref[...] = jnp.dot(x_ref[...], w_ref[...], precision=PREC,
                         preferred_element_type=jnp.float32)


def _seer_kernel(q_ref, k_ref, v_ref, o_ref, e0_scr, e1_scr, e2_scr, e3_scr):
    escr = (e0_scr, e1_scr, e2_scr, e3_scr)
    qt = pl.program_id(1)
    diag = _iota((TQ, CH), 0) >= _iota((TQ, CH), 1)
    rq_t = (_iota((GB, TQ), 1) // BLK
            == _iota((GB, TQ), 0)).astype(jnp.bfloat16)
    rq = (_iota((TQ, GB), 0) // BLK == _iota((TQ, GB), 1)).astype(jnp.float32)
    qs = [q_ref[:, h * HD:(h + 1) * HD] for h in range(4)]

    def _chunk(c, carry, masked):
        acc = list(carry)
        kc = k_ref[pl.ds(c * CH, CH), :]
        rk_c = (((c * CH + _iota((CH, NB), 0)) // BLK)
                == _iota((CH, NB), 1)).astype(jnp.bfloat16)
        for h in range(4):
            kh = kc[:, (h // 2) * HD:(h // 2 + 1) * HD]
            s = jax.lax.dot_general(
                qs[h], kh, (((1,), (1,)), ((), ())), precision=PREC,
                preferred_element_type=jnp.float32) * SCALE
            if masked:
                s = jnp.where(diag, s, NEG)
            e = jnp.exp(s)
            escr[h][c] = e
            acc[h] = acc[h] + _split_dot_xi(e, rk_c)
        return tuple(acc)

    z_a = jnp.zeros((TQ, NB), jnp.float32)
    carry = jax.lax.fori_loop(0, qt, lambda c, cy: _chunk(c, cy, False),
                              (z_a,) * 4)
    aa = _chunk(qt, carry, True)
    ll = [jnp.sum(a, axis=-1, keepdims=True) for a in aa]
    pp = [_split_dot_ix(rq_t, aa[h] / ll[h]) for h in range(4)]
    pooled = jnp.concatenate([jnp.maximum(pp[0], pp[1]),
                              jnp.maximum(pp[2], pp[3])], axis=0)
    qb = qt * GB + _iota((2 * GB, NB), 0) % GB
    kb = _iota((2 * GB, NB), 1)
    pc = jnp.where(kb <= qb, pooled, -1.0)
    cnt = jnp.zeros((2 * GB, NB), jnp.float32)
    one = jnp.ones((2 * GB, NB), jnp.float32)
    zero = jnp.zeros((2 * GB, NB), jnp.float32)
    for shift in range(1, NB):
        r = jnp.roll(pc, shift, axis=1)
        cnt += jnp.where(r > pc, one, zero)
        cnt += jnp.where((r == pc) & (kb >= shift), one, zero)
    sel = cnt < BUDGET
    window = ((qb - kb) < SWIN) & (kb <= qb)
    allowed = sel | window | (kb == 0)
    b01 = jnp.where(allowed, 1.0, 0.0)
    b01a = b01[:GB]
    b01b = b01[GB:]
    b01ea = _dot32(rq, b01a)
    b01eb = _dot32(rq, b01b)
    l2 = [jnp.sum(aa[0] * b01ea, axis=-1, keepdims=True),
          jnp.sum(aa[1] * b01ea, axis=-1, keepdims=True),
          jnp.sum(aa[2] * b01eb, axis=-1, keepdims=True),
          jnp.sum(aa[3] * b01eb, axis=-1, keepdims=True)]

    def body2(c, carry):
        os_ = list(carry)
        vc = v_ref[pl.ds(c * CH, CH), :]
        rkt_c = (_iota((NB, CH), 0)
                 == (c * CH + _iota((NB, CH), 1)) // BLK).astype(jnp.float32)
        bma = _dot32(rq, _dot32(b01a, rkt_c))
        bmb = _dot32(rq, _dot32(b01b, rkt_c))
        for h in range(4):
            bm = bma if h < 2 else bmb
            vh = vc[:, (h // 2) * HD:(h // 2 + 1) * HD]
            os_[h] = os_[h] + _dot32(escr[h][c] * bm, vh)
        return tuple(os_)

    z_o = jnp.zeros((TQ, HD), jnp.float32)
    oo = jax.lax.fori_loop(0, qt + 1, body2, (z_o,) * 4)
    o_ref[...] = jnp.concatenate([oo[h] / l2[h] for h in range(4)], axis=-1)


TS = 256


def _qkv_call(x, wq, wk, wv, cs, sn, qn, kn):
    return pl.pallas_call(
        _qkv_kernel,
        grid=(S // TS,),
        in_specs=[
            pl.BlockSpec((TS, D), lambda s: (s, 0)),
            pl.BlockSpec((D, H * HD), lambda s: (0, 0)),
            pl.BlockSpec((D, KV * HD), lambda s: (0, 0)),
            pl.BlockSpec((D, KV * HD), lambda s: (0, 0)),
            pl.BlockSpec((TS, HD), lambda s: (s, 0)),
            pl.BlockSpec((TS, HD), lambda s: (s, 0)),
            pl.BlockSpec((1, HD), lambda s: (0, 0)),
            pl.BlockSpec((1, HD), lambda s: (0, 0)),
        ],
        out_specs=pl.BlockSpec((TS, (H + 2 * KV) * HD), lambda s: (s, 0)),
        out_shape=jax.ShapeDtypeStruct((S, (H + 2 * KV) * HD), jnp.float32),
    )(x, wq, wk, wv, cs, sn, qn, kn)


def _mm_call(x, w, bm, bn):
    gm, gn = x.shape[0] // bm, w.shape[1] // bn
    return pl.pallas_call(
        _mm_kernel,
        grid=(gn, gm),
        in_specs=[
            pl.BlockSpec((bm, x.shape[1]), lambda n, m: (m, 0)),
            pl.BlockSpec((w.shape[0], bn), lambda n, m: (0, n)),
        ],
        out_specs=pl.BlockSpec((bm, bn), lambda n, m: (m, n)),
        out_shape=jax.ShapeDtypeStruct((x.shape[0], w.shape[1]), jnp.float32),
    )(x, w)


def _seer_call(qkv):
    return pl.pallas_call(
        _seer_kernel,
        grid=(KV // 2, QT),
        in_specs=[
            pl.BlockSpec((TQ, 4 * HD), lambda j, qt: (qt, j)),
            pl.BlockSpec((S, 2 * HD), lambda j, qt: (0, (H * HD) // (2 * HD) + j)),
            pl.BlockSpec((S, 2 * HD),
                         lambda j, qt: (0, ((H + KV) * HD) // (2 * HD) + j)),
        ],
        out_specs=pl.BlockSpec((TQ, 4 * HD), lambda j, qt: (qt, j)),
        out_shape=jax.ShapeDtypeStruct((S, H * HD), jnp.float32),
        scratch_shapes=[
            pltpu.VMEM((QT, TQ, CH), jnp.float32),
            pltpu.VMEM((QT, TQ, CH), jnp.float32),
            pltpu.VMEM((QT, TQ, CH), jnp.float32),
            pltpu.VMEM((QT, TQ, CH), jnp.float32),
        ],
    )(qkv, qkv, qkv)


def kernel(hidden_states, cos, sin, Wq, Wk, Wv, Wo, q_norm_w, k_norm_w):
    x = hidden_states[0]
    cs = cos[0]
    sn = sin[0]
    qn = q_norm_w.reshape(1, HD)
    kn = k_norm_w.reshape(1, HD)
    qkv = _qkv_call(x, Wq, Wk, Wv, cs, sn, qn, kn)
    att = _seer_call(qkv)
    out = _mm_call(att, Wo, 1024, 1024)
    return out[None]


# speedup vs baseline: 1.2559x; 1.0119x over previous
import jax
import jax.numpy as jnp
from jax.experimental import pallas as pl
from jax.experimental.pallas import tpu as pltpu

S, D = 2048, 2048
H, KV, HD = 16, 8, 128
G = H // KV
BLK = 64
NB = S // BLK
BUDGET = 16
SWIN = 4
SCALE = HD ** -0.5
EPS = 1e-6
NEG = -1e30

TQ = 512
QT = S // TQ
GB = TQ // BLK

CH = 512

PREC = jax.lax.Precision.DEFAULT
HI = jax.lax.Precision.HIGHEST


def _iota(shape, dim):
    return jax.lax.broadcasted_iota(jnp.int32, shape, dim)


def _dot32(a, b):
    return jnp.dot(a, b, precision=PREC, preferred_element_type=jnp.float32)


def _split_dot_xi(x, ind_bf):
    xh = x.astype(jnp.bfloat16)
    xl = (x - xh.astype(jnp.float32)).astype(jnp.bfloat16)
    return (jnp.dot(xh, ind_bf, preferred_element_type=jnp.float32)
            + jnp.dot(xl, ind_bf, preferred_element_type=jnp.float32))


def _split_dot_ix(ind_bf, x):
    xh = x.astype(jnp.bfloat16)
    xl = (x - xh.astype(jnp.float32)).astype(jnp.bfloat16)
    return (jnp.dot(ind_bf, xh, preferred_element_type=jnp.float32)
            + jnp.dot(ind_bf, xl, preferred_element_type=jnp.float32))


def _norm_rope(yh, nw, cs, sn):
    var = jnp.mean(yh * yh, axis=-1, keepdims=True)
    n = (yh * jax.lax.rsqrt(var + EPS)) * nw
    rot = jnp.concatenate([-n[:, HD // 2:], n[:, :HD // 2]], axis=-1)
    return n * cs + rot * sn


def _qkv_kernel(x_ref, wq_ref, wk_ref, wv_ref, cs_ref, sn_ref,
                qn_ref, kn_ref, o_ref):
    x = x_ref[...]
    yq = jnp.dot(x, wq_ref[...], precision=PREC,
                 preferred_element_type=jnp.float32)
    yk = jnp.dot(x, wk_ref[...], precision=PREC,
                 preferred_element_type=jnp.float32)
    yv = jnp.dot(x, wv_ref[...], precision=PREC,
                 preferred_element_type=jnp.float32)
    cs = cs_ref[...]
    sn = sn_ref[...]
    qn = qn_ref[...]
    kn = kn_ref[...]
    cols = []
    for h in range(H):
        cols.append(_norm_rope(yq[:, h * HD:(h + 1) * HD], qn, cs, sn))
    for h in range(KV):
        cols.append(_norm_rope(yk[:, h * HD:(h + 1) * HD], kn, cs, sn))
    cols.append(yv)
    o_ref[...] = jnp.concatenate(cols, axis=-1)


def _mm_kernel(x_ref, w_ref, o_ref):
    o_ref[...] = jnp.dot(x_ref[...], w_ref[...], precision=PREC,
                         preferred_element_type=jnp.float32)


def _seer_kernel(q_ref, k_ref, v_ref, o_ref, e0_scr, e1_scr, e2_scr, e3_scr):
    escr = (e0_scr, e1_scr, e2_scr, e3_scr)
    qt = pl.program_id(1)
    diag = _iota((TQ, CH), 0) >= _iota((TQ, CH), 1)
    rq_t = (_iota((GB, TQ), 1) // BLK
            == _iota((GB, TQ), 0)).astype(jnp.bfloat16)
    rq = (_iota((TQ, GB), 0) // BLK == _iota((TQ, GB), 1)).astype(jnp.float32)
    qs = [q_ref[:, h * HD:(h + 1) * HD] for h in range(4)]

    def _chunk(c, carry, masked):
        acc = list(carry)
        kc = k_ref[pl.ds(c * CH, CH), :]
        rk_c = (((c * CH + _iota((CH, NB), 0)) // BLK)
                == _iota((CH, NB), 1)).astype(jnp.bfloat16)
        for h in range(4):
            kh = kc[:, (h // 2) * HD:(h // 2 + 1) * HD]
            s = jax.lax.dot_general(
                qs[h], kh, (((1,), (1,)), ((), ())), precision=PREC,
                preferred_element_type=jnp.float32) * SCALE
            if masked:
                s = jnp.where(diag, s, NEG)
            e = jnp.exp(s)
            escr[h][c] = e
            acc[h] = acc[h] + _split_dot_xi(e, rk_c)
        return tuple(acc)

    z_a = jnp.zeros((TQ, NB), jnp.float32)
    carry = jax.lax.fori_loop(0, qt, lambda c, cy: _chunk(c, cy, False),
                              (z_a,) * 4)
    aa = _chunk(qt, carry, True)
    ll = [jnp.sum(a, axis=-1, keepdims=True) for a in aa]
    pp = [_split_dot_ix(rq_t, aa[h] / ll[h]) for h in range(4)]
    pooled = jnp.concatenate([jnp.maximum(pp[0], pp[1]),
                              jnp.maximum(pp[2], pp[3])], axis=0)
    qb = qt * GB + _iota((2 * GB, NB), 0) % GB
    kb = _iota((2 * GB, NB), 1)
    pc = jnp.where(kb <= qb, pooled, -1.0)
    cnt = jnp.zeros((2 * GB, NB), jnp.float32)
    one = jnp.ones((2 * GB, NB), jnp.float32)
    zero = jnp.zeros((2 * GB, NB), jnp.float32)
    for shift in range(1, NB):
        r = jnp.roll(pc, shift, axis=1)
        cnt += jnp.where(r > pc, one, zero)
        cnt += jnp.where((r == pc) & (kb >= shift), one, zero)
    sel = cnt < BUDGET
    window = ((qb - kb) < SWIN) & (kb <= qb)
    allowed = sel | window | (kb == 0)
    b01 = jnp.where(allowed, 1.0, 0.0)
    b01a = b01[:GB]
    b01b = b01[GB:]
    b01ea = _dot32(rq, b01a)
    b01eb = _dot32(rq, b01b)
    l2 = [jnp.sum(aa[0] * b01ea, axis=-1, keepdims=True),
          jnp.sum(aa[1] * b01ea, axis=-1, keepdims=True),
          jnp.sum(aa[2] * b01eb, axis=-1, keepdims=True),
          jnp.sum(aa[3] * b01eb, axis=-1, keepdims=True)]

    def body2(c, carry):
        os_ = list(carry)
        vc = v_ref[pl.ds(c * CH, CH), :]
        rkt_c = (_iota((NB, CH), 0)
                 == (c * CH + _iota((NB, CH), 1)) // BLK).astype(jnp.float32)
        bma = _dot32(rq, _dot32(b01a, rkt_c))
        bmb = _dot32(rq, _dot32(b01b, rkt_c))
        for h in range(4):
            bm = bma if h < 2 else bmb
            vh = vc[:, (h // 2) * HD:(h // 2 + 1) * HD]
            os_[h] = os_[h] + _dot32(escr[h][c] * bm, vh)
        return tuple(os_)

    z_o = jnp.zeros((TQ, HD), jnp.float32)
    oo = jax.lax.fori_loop(0, qt + 1, body2, (z_o,) * 4)
    o_ref[...] = jnp.concatenate([oo[h] / l2[h] for h in range(4)], axis=-1)


TS = 256


def _qkv_call(x, wq, wk, wv, cs, sn, qn, kn):
    return pl.pallas_call(
        _qkv_kernel,
        grid=(S // TS,),
        in_specs=[
            pl.BlockSpec((TS, D), lambda s: (s, 0)),
            pl.BlockSpec((D, H * HD), lambda s: (0, 0)),
            pl.BlockSpec((D, KV * HD), lambda s: (0, 0)),
            pl.BlockSpec((D, KV * HD), lambda s: (0, 0)),
            pl.BlockSpec((TS, HD), lambda s: (s, 0)),
            pl.BlockSpec((TS, HD), lambda s: (s, 0)),
            pl.BlockSpec((1, HD), lambda s: (0, 0)),
            pl.BlockSpec((1, HD), lambda s: (0, 0)),
        ],
        out_specs=pl.BlockSpec((TS, (H + 2 * KV) * HD), lambda s: (s, 0)),
        out_shape=jax.ShapeDtypeStruct((S, (H + 2 * KV) * HD), jnp.float32),
    )(x, wq, wk, wv, cs, sn, qn, kn)


def _mm_call(x, w, bm):
    return pl.pallas_call(
        _mm_kernel,
        grid=(x.shape[0] // bm,),
        in_specs=[
            pl.BlockSpec((bm, x.shape[1]), lambda m: (m, 0)),
            pl.BlockSpec((w.shape[0], w.shape[1]), lambda m: (0, 0)),
        ],
        out_specs=pl.BlockSpec((bm, w.shape[1]), lambda m: (m, 0)),
        out_shape=jax.ShapeDtypeStruct((x.shape[0], w.shape[1]), jnp.float32),
    )(x, w)


def _seer_call(qkv):
    return pl.pallas_call(
        _seer_kernel,
        grid=(KV // 2, QT),
        in_specs=[
            pl.BlockSpec((TQ, 4 * HD), lambda j, qt: (qt, j)),
            pl.BlockSpec((S, 2 * HD), lambda j, qt: (0, (H * HD) // (2 * HD) + j)),
            pl.BlockSpec((S, 2 * HD),
                         lambda j, qt: (0, ((H + KV) * HD) // (2 * HD) + j)),
        ],
        out_specs=pl.BlockSpec((TQ, 4 * HD), lambda j, qt: (qt, j)),
        out_shape=jax.ShapeDtypeStruct((S, H * HD), jnp.float32),
        scratch_shapes=[
            pltpu.VMEM((QT, TQ, CH), jnp.float32),
            pltpu.VMEM((QT, TQ, CH), jnp.float32),
            pltpu.VMEM((QT, TQ, CH), jnp.float32),
            pltpu.VMEM((QT, TQ, CH), jnp.float32),
        ],
    )(qkv, qkv, qkv)


def kernel(hidden_states, cos, sin, Wq, Wk, Wv, Wo, q_norm_w, k_norm_w):
    x = hidden_states[0]
    cs = cos[0]
    sn = sin[0]
    qn = q_norm_w.reshape(1, HD)
    kn = k_norm_w.reshape(1, HD)
    qkv = _qkv_call(x, Wq, Wk, Wv, cs, sn, qn, kn)
    att = _seer_call(qkv)
    out = _mm_call(att, Wo, 512)
    return out[None]


# speedup vs baseline: 1.3377x; 1.0651x over previous
import jax
import jax.numpy as jnp
from jax.experimental import pallas as pl
from jax.experimental.pallas import tpu as pltpu

S, D = 2048, 2048
H, KV, HD = 16, 8, 128
G = H // KV
BLK = 64
NB = S // BLK
BUDGET = 16
SWIN = 4
SCALE = HD ** -0.5
EPS = 1e-6
NEG = -1e30

TQ = 512
QT = S // TQ
GB = TQ // BLK

CH = 512

PREC = jax.lax.Precision.DEFAULT
HI = jax.lax.Precision.HIGHEST


def _iota(shape, dim):
    return jax.lax.broadcasted_iota(jnp.int32, shape, dim)


def _dot32(a, b):
    return jnp.dot(a, b, precision=PREC, preferred_element_type=jnp.float32)


def _split_dot_xi(x, ind_bf):
    xh = x.astype(jnp.bfloat16)
    xl = (x - xh.astype(jnp.float32)).astype(jnp.bfloat16)
    return (jnp.dot(xh, ind_bf, preferred_element_type=jnp.float32)
            + jnp.dot(xl, ind_bf, preferred_element_type=jnp.float32))


def _split_dot_ix(ind_bf, x):
    xh = x.astype(jnp.bfloat16)
    xl = (x - xh.astype(jnp.float32)).astype(jnp.bfloat16)
    return (jnp.dot(ind_bf, xh, preferred_element_type=jnp.float32)
            + jnp.dot(ind_bf, xl, preferred_element_type=jnp.float32))


def _norm_rope(yh, nw, cs, sn):
    var = jnp.mean(yh * yh, axis=-1, keepdims=True)
    n = (yh * jax.lax.rsqrt(var + EPS)) * nw
    rot = jnp.concatenate([-n[:, HD // 2:], n[:, :HD // 2]], axis=-1)
    return n * cs + rot * sn


def _qkv_kernel(x_ref, wq_ref, wk_ref, wv_ref, cs_ref, sn_ref,
                qn_ref, kn_ref, o_ref):
    x = x_ref[...]
    yq = jnp.dot(x, wq_ref[...], precision=PREC,
                 preferred_element_type=jnp.float32)
    yk = jnp.dot(x, wk_ref[...], precision=PREC,
                 preferred_element_type=jnp.float32)
    yv = jnp.dot(x, wv_ref[...], precision=PREC,
                 preferred_element_type=jnp.float32)
    cs = cs_ref[...]
    sn = sn_ref[...]
    qn = qn_ref[...]
    kn = kn_ref[...]
    cols = []
    for h in range(H):
        cols.append(_norm_rope(yq[:, h * HD:(h + 1) * HD], qn, cs, sn))
    for h in range(KV):
        cols.append(_norm_rope(yk[:, h * HD:(h + 1) * HD], kn, cs, sn))
    cols.append(yv)
    o_ref[...] = jnp.concatenate(cols, axis=-1)


def _mm_kernel(x_ref, w_ref, o_ref):
    o_ref[...] = jnp.dot(x_ref[...], w_ref[...], precision=PREC,
                         preferred_element_type=jnp.float32)


def _seer_kernel(q_ref, k_ref, v_ref, o_ref, e0_scr, e1_scr, e2_scr, e3_scr):
    escr = (e0_scr, e1_scr, e2_scr, e3_scr)
    qt = pl.program_id(1)
    diag = _iota((TQ, CH), 0) >= _iota((TQ, CH), 1)
    rq_t = (_iota((GB, TQ), 1) // BLK
            == _iota((GB, TQ), 0)).astype(jnp.bfloat16)
    rq = (_iota((TQ, GB), 0) // BLK == _iota((TQ, GB), 1)).astype(jnp.float32)
    qs = [q_ref[:, h * HD:(h + 1) * HD] for h in range(4)]

    def _chunk(c, carry, masked):
        acc = list(carry)
        kc = k_ref[pl.ds(c * CH, CH), :]
        rk_c = (((c * CH + _iota((CH, NB), 0)) // BLK)
                == _iota((CH, NB), 1)).astype(jnp.bfloat16)
        for h in range(4):
            kh = kc[:, (h // 2) * HD:(h // 2 + 1) * HD]
            s = jax.lax.dot_general(
                qs[h], kh, (((1,), (1,)), ((), ())), precision=PREC,
                preferred_element_type=jnp.float32) * SCALE
            if masked:
                s = jnp.where(diag, s, NEG)
            e = jnp.exp(s)
            escr[h][c] = e
            acc[h] = acc[h] + _split_dot_xi(e, rk_c)
        return tuple(acc)

    z_a = jnp.zeros((TQ, NB), jnp.float32)
    carry = jax.lax.fori_loop(0, qt, lambda c, cy: _chunk(c, cy, False),
                              (z_a,) * 4)
    aa = _chunk(qt, carry, True)
    ll = [jnp.sum(a, axis=-1, keepdims=True) for a in aa]
    pp = [_split_dot_ix(rq_t, aa[h] / ll[h]) for h in range(4)]
    pooled = jnp.concatenate([jnp.maximum(pp[0], pp[1]),
                              jnp.maximum(pp[2], pp[3])], axis=0)
    qb = qt * GB + _iota((2 * GB, NB), 0) % GB
    kb = _iota((2 * GB, NB), 1)
    pc = jnp.where(kb <= qb, pooled, -1.0)
    cnt = jnp.zeros((2 * GB, NB), jnp.float32)
    one = jnp.ones((2 * GB, NB), jnp.float32)
    zero = jnp.zeros((2 * GB, NB), jnp.float32)
    for shift in range(1, NB):
        r = jnp.roll(pc, shift, axis=1)
        cnt += jnp.where(r > pc, one, zero)
        cnt += jnp.where((r == pc) & (kb >= shift), one, zero)
    sel = cnt < BUDGET
    window = ((qb - kb) < SWIN) & (kb <= qb)
    allowed = sel | window | (kb == 0)
    b01 = jnp.where(allowed, 1.0, 0.0)
    b01a = b01[:GB]
    b01b = b01[GB:]
    b01ea = _dot32(rq, b01a)
    b01eb = _dot32(rq, b01b)
    l2 = [jnp.sum(aa[0] * b01ea, axis=-1, keepdims=True),
          jnp.sum(aa[1] * b01ea, axis=-1, keepdims=True),
          jnp.sum(aa[2] * b01eb, axis=-1, keepdims=True),
          jnp.sum(aa[3] * b01eb, axis=-1, keepdims=True)]

    def body2(c, carry):
        os_ = list(carry)
        vc = v_ref[pl.ds(c * CH, CH), :]
        rkt_c = (_iota((NB, CH), 0)
                 == (c * CH + _iota((NB, CH), 1)) // BLK).astype(jnp.float32)
        bma = jnp.repeat(_dot32(b01a, rkt_c), BLK, axis=0)
        bmb = jnp.repeat(_dot32(b01b, rkt_c), BLK, axis=0)
        for h in range(4):
            bm = bma if h < 2 else bmb
            vh = vc[:, (h // 2) * HD:(h // 2 + 1) * HD]
            os_[h] = os_[h] + _dot32(escr[h][c] * bm, vh)
        return tuple(os_)

    z_o = jnp.zeros((TQ, HD), jnp.float32)
    oo = jax.lax.fori_loop(0, qt + 1, body2, (z_o,) * 4)
    o_ref[...] = jnp.concatenate([oo[h] / l2[h] for h in range(4)], axis=-1)


TS = 256


def _qkv_call(x, wq, wk, wv, cs, sn, qn, kn):
    return pl.pallas_call(
        _qkv_kernel,
        grid=(S // TS,),
        in_specs=[
            pl.BlockSpec((TS, D), lambda s: (s, 0)),
            pl.BlockSpec((D, H * HD), lambda s: (0, 0)),
            pl.BlockSpec((D, KV * HD), lambda s: (0, 0)),
            pl.BlockSpec((D, KV * HD), lambda s: (0, 0)),
            pl.BlockSpec((TS, HD), lambda s: (s, 0)),
            pl.BlockSpec((TS, HD), lambda s: (s, 0)),
            pl.BlockSpec((1, HD), lambda s: (0, 0)),
            pl.BlockSpec((1, HD), lambda s: (0, 0)),
        ],
        out_specs=pl.BlockSpec((TS, (H + 2 * KV) * HD), lambda s: (s, 0)),
        out_shape=jax.ShapeDtypeStruct((S, (H + 2 * KV) * HD), jnp.float32),
    )(x, wq, wk, wv, cs, sn, qn, kn)


def _mm_call(x, w, bm):
    return pl.pallas_call(
        _mm_kernel,
        grid=(x.shape[0] // bm,),
        in_specs=[
            pl.BlockSpec((bm, x.shape[1]), lambda m: (m, 0)),
            pl.BlockSpec((w.shape[0], w.shape[1]), lambda m: (0, 0)),
        ],
        out_specs=pl.BlockSpec((bm, w.shape[1]), lambda m: (m, 0)),
        out_shape=jax.ShapeDtypeStruct((x.shape[0], w.shape[1]), jnp.float32),
    )(x, w)


def _seer_call(qkv):
    return pl.pallas_call(
        _seer_kernel,
        grid=(KV // 2, QT),
        in_specs=[
            pl.BlockSpec((TQ, 4 * HD), lambda j, qt: (qt, j)),
            pl.BlockSpec((S, 2 * HD), lambda j, qt: (0, (H * HD) // (2 * HD) + j)),
            pl.BlockSpec((S, 2 * HD),
                         lambda j, qt: (0, ((H + KV) * HD) // (2 * HD) + j)),
        ],
        out_specs=pl.BlockSpec((TQ, 4 * HD), lambda j, qt: (qt, j)),
        out_shape=jax.ShapeDtypeStruct((S, H * HD), jnp.float32),
        scratch_shapes=[
            pltpu.VMEM((QT, TQ, CH), jnp.float32),
            pltpu.VMEM((QT, TQ, CH), jnp.float32),
            pltpu.VMEM((QT, TQ, CH), jnp.float32),
            pltpu.VMEM((QT, TQ, CH), jnp.float32),
        ],
    )(qkv, qkv, qkv)


def kernel(hidden_states, cos, sin, Wq, Wk, Wv, Wo, q_norm_w, k_norm_w):
    x = hidden_states[0]
    cs = cos[0]
    sn = sin[0]
    qn = q_norm_w.reshape(1, HD)
    kn = k_norm_w.reshape(1, HD)
    qkv = _qkv_call(x, Wq, Wk, Wv, cs, sn, qn, kn)
    att = _seer_call(qkv)
    out = _mm_call(att, Wo, 512)
    return out[None]


# speedup vs baseline: 1.3425x; 1.0036x over previous
import jax
import jax.numpy as jnp
from jax.experimental import pallas as pl
from jax.experimental.pallas import tpu as pltpu

S, D = 2048, 2048
H, KV, HD = 16, 8, 128
G = H // KV
BLK = 64
NB = S // BLK
BUDGET = 16
SWIN = 4
SCALE = HD ** -0.5
EPS = 1e-6
NEG = -1e30

TQ = 512
QT = S // TQ
GB = TQ // BLK

CH = 512

PREC = jax.lax.Precision.DEFAULT
HI = jax.lax.Precision.HIGHEST


def _iota(shape, dim):
    return jax.lax.broadcasted_iota(jnp.int32, shape, dim)


def _dot32(a, b):
    return jnp.dot(a, b, precision=PREC, preferred_element_type=jnp.float32)


def _split_dot_xi(x, ind_bf):
    xh = x.astype(jnp.bfloat16)
    xl = (x - xh.astype(jnp.float32)).astype(jnp.bfloat16)
    return (jnp.dot(xh, ind_bf, preferred_element_type=jnp.float32)
            + jnp.dot(xl, ind_bf, preferred_element_type=jnp.float32))


def _split_dot_ix(ind_bf, x):
    xh = x.astype(jnp.bfloat16)
    xl = (x - xh.astype(jnp.float32)).astype(jnp.bfloat16)
    return (jnp.dot(ind_bf, xh, preferred_element_type=jnp.float32)
            + jnp.dot(ind_bf, xl, preferred_element_type=jnp.float32))


def _norm_rope(yh, nw, cs, sn):
    var = jnp.mean(yh * yh, axis=-1, keepdims=True)
    n = (yh * jax.lax.rsqrt(var + EPS)) * nw
    rot = jnp.concatenate([-n[:, HD // 2:], n[:, :HD // 2]], axis=-1)
    return n * cs + rot * sn


def _qkv_kernel(x_ref, wq_ref, wk_ref, wv_ref, cs_ref, sn_ref,
                qn_ref, kn_ref, o_ref):
    x = x_ref[...]
    yq = jnp.dot(x, wq_ref[...], precision=PREC,
                 preferred_element_type=jnp.float32)
    yk = jnp.dot(x, wk_ref[...], precision=PREC,
                 preferred_element_type=jnp.float32)
    yv = jnp.dot(x, wv_ref[...], precision=PREC,
                 preferred_element_type=jnp.float32)
    cs = cs_ref[...]
    sn = sn_ref[...]
    qn = qn_ref[...]
    kn = kn_ref[...]
    cols = []
    for h in range(H):
        cols.append(_norm_rope(yq[:, h * HD:(h + 1) * HD], qn, cs, sn))
    for h in range(KV):
        cols.append(_norm_rope(yk[:, h * HD:(h + 1) * HD], kn, cs, sn))
    cols.append(yv)
    o_ref[...] = jnp.concatenate(cols, axis=-1)


def _mm_kernel(x_ref, w_ref, o_ref):
    o_ref[...] = jnp.dot(x_ref[...], w_ref[...], precision=PREC,
                         preferred_element_type=jnp.float32)


def _seer_kernel(q_ref, k_ref, v_ref, o_ref, e0_scr, e1_scr, e2_scr, e3_scr):
    escr = (e0_scr, e1_scr, e2_scr, e3_scr)
    qt = pl.program_id(1)
    diag = _iota((TQ, CH), 0) >= _iota((TQ, CH), 1)
    rq_t = (_iota((GB, TQ), 1) // BLK
            == _iota((GB, TQ), 0)).astype(jnp.bfloat16)
    qs = [q_ref[:, h * HD:(h + 1) * HD] for h in range(4)]

    def _chunk(c, carry, masked):
        acc = list(carry)
        kc = k_ref[pl.ds(c * CH, CH), :]
        rk_c = (((c * CH + _iota((CH, NB), 0)) // BLK)
                == _iota((CH, NB), 1)).astype(jnp.bfloat16)
        for h in range(4):
            kh = kc[:, (h // 2) * HD:(h // 2 + 1) * HD]
            s = jax.lax.dot_general(
                qs[h], kh, (((1,), (1,)), ((), ())), precision=PREC,
                preferred_element_type=jnp.float32) * SCALE
            if masked:
                s = jnp.where(diag, s, NEG)
            e = jnp.exp(s)
            escr[h][c] = e
            acc[h] = acc[h] + _split_dot_xi(e, rk_c)
        return tuple(acc)

    z_a = jnp.zeros((TQ, NB), jnp.float32)
    carry = jax.lax.fori_loop(0, qt, lambda c, cy: _chunk(c, cy, False),
                              (z_a,) * 4)
    aa = _chunk(qt, carry, True)
    ll = [jnp.sum(a, axis=-1, keepdims=True) for a in aa]
    pp = [_split_dot_ix(rq_t, aa[h] / ll[h]) for h in range(4)]
    pooled = jnp.concatenate([jnp.maximum(pp[0], pp[1]),
                              jnp.maximum(pp[2], pp[3])], axis=0)
    qb = qt * GB + _iota((2 * GB, NB), 0) % GB
    kb = _iota((2 * GB, NB), 1)
    pc = jnp.where(kb <= qb, pooled, -1.0)
    cnt = jnp.zeros((2 * GB, NB), jnp.float32)
    one = jnp.ones((2 * GB, NB), jnp.float32)
    zero = jnp.zeros((2 * GB, NB), jnp.float32)
    for shift in range(1, NB):
        r = jnp.roll(pc, shift, axis=1)
        cnt += jnp.where(r > pc, one, zero)
        cnt += jnp.where((r == pc) & (kb >= shift), one, zero)
    sel = cnt < BUDGET
    window = ((qb - kb) < SWIN) & (kb <= qb)
    allowed = sel | window | (kb == 0)
    b01 = jnp.where(allowed, 1.0, 0.0)
    b01a = b01[:GB]
    b01b = b01[GB:]
    b01ea = jnp.repeat(b01a, BLK, axis=0)
    b01eb = jnp.repeat(b01b, BLK, axis=0)
    l2 = [jnp.sum(aa[0] * b01ea, axis=-1, keepdims=True),
          jnp.sum(aa[1] * b01ea, axis=-1, keepdims=True),
          jnp.sum(aa[2] * b01eb, axis=-1, keepdims=True),
          jnp.sum(aa[3] * b01eb, axis=-1, keepdims=True)]

    def body2(c, carry):
        os_ = list(carry)
        vc = v_ref[pl.ds(c * CH, CH), :]
        rkt_c = (_iota((NB, CH), 0)
                 == (c * CH + _iota((NB, CH), 1)) // BLK).astype(jnp.float32)
        bma = jnp.repeat(_dot32(b01a, rkt_c), BLK, axis=0)
        bmb = jnp.repeat(_dot32(b01b, rkt_c), BLK, axis=0)
        for h in range(4):
            bm = bma if h < 2 else bmb
            vh = vc[:, (h // 2) * HD:(h // 2 + 1) * HD]
            os_[h] = os_[h] + _dot32(escr[h][c] * bm, vh)
        return tuple(os_)

    z_o = jnp.zeros((TQ, HD), jnp.float32)
    oo = jax.lax.fori_loop(0, qt + 1, body2, (z_o,) * 4)
    o_ref[...] = jnp.concatenate([oo[h] / l2[h] for h in range(4)], axis=-1)


TS = 256


def _qkv_call(x, wq, wk, wv, cs, sn, qn, kn):
    return pl.pallas_call(
        _qkv_kernel,
        grid=(S // TS,),
        in_specs=[
            pl.BlockSpec((TS, D), lambda s: (s, 0)),
            pl.BlockSpec((D, H * HD), lambda s: (0, 0)),
            pl.BlockSpec((D, KV * HD), lambda s: (0, 0)),
            pl.BlockSpec((D, KV * HD), lambda s: (0, 0)),
            pl.BlockSpec((TS, HD), lambda s: (s, 0)),
            pl.BlockSpec((TS, HD), lambda s: (s, 0)),
            pl.BlockSpec((1, HD), lambda s: (0, 0)),
            pl.BlockSpec((1, HD), lambda s: (0, 0)),
        ],
        out_specs=pl.BlockSpec((TS, (H + 2 * KV) * HD), lambda s: (s, 0)),
        out_shape=jax.ShapeDtypeStruct((S, (H + 2 * KV) * HD), jnp.float32),
    )(x, wq, wk, wv, cs, sn, qn, kn)


def _mm_call(x, w, bm):
    return pl.pallas_call(
        _mm_kernel,
        grid=(x.shape[0] // bm,),
        in_specs=[
            pl.BlockSpec((bm, x.shape[1]), lambda m: (m, 0)),
            pl.BlockSpec((w.shape[0], w.shape[1]), lambda m: (0, 0)),
        ],
        out_specs=pl.BlockSpec((bm, w.shape[1]), lambda m: (m, 0)),
        out_shape=jax.ShapeDtypeStruct((x.shape[0], w.shape[1]), jnp.float32),
    )(x, w)


def _seer_call(qkv):
    return pl.pallas_call(
        _seer_kernel,
        grid=(KV // 2, QT),
        in_specs=[
            pl.BlockSpec((TQ, 4 * HD), lambda j, qt: (qt, j)),
            pl.BlockSpec((S, 2 * HD), lambda j, qt: (0, (H * HD) // (2 * HD) + j)),
            pl.BlockSpec((S, 2 * HD),
                         lambda j, qt: (0, ((H + KV) * HD) // (2 * HD) + j)),
        ],
        out_specs=pl.BlockSpec((TQ, 4 * HD), lambda j, qt: (qt, j)),
        out_shape=jax.ShapeDtypeStruct((S, H * HD), jnp.float32),
        scratch_shapes=[
            pltpu.VMEM((QT, TQ, CH), jnp.float32),
            pltpu.VMEM((QT, TQ, CH), jnp.float32),
            pltpu.VMEM((QT, TQ, CH), jnp.float32),
            pltpu.VMEM((QT, TQ, CH), jnp.float32),
        ],
    )(qkv, qkv, qkv)


def kernel(hidden_states, cos, sin, Wq, Wk, Wv, Wo, q_norm_w, k_norm_w):
    x = hidden_states[0]
    cs = cos[0]
    sn = sin[0]
    qn = q_norm_w.reshape(1, HD)
    kn = k_norm_w.reshape(1, HD)
    qkv = _qkv_call(x, Wq, Wk, Wv, cs, sn, qn, kn)
    att = _seer_call(qkv)
    out = _mm_call(att, Wo, 512)
    return out[None]
